# baseline (speedup 1.0000x reference)
import functools

import jax
import jax.numpy as jnp
import numpy as np
from jax import lax
from jax.experimental import pallas as pl
from jax.experimental.pallas import tpu as pltpu
from jax.experimental.pallas import tpu_sc as plsc

N0, N1, N2 = 10000, 5000, 2000
E0, E1 = 320000, 160000
D = 128

NC, NS = 2, 16
NW = NC * NS
C = 128

N1_PAD = 5120
N2_PAD = 2048
def _round_chunks(e):
    n = (e + NW * C - 1) // (NW * C)
    return n + (-n) % 3

E0_PAD = _round_chunks(E0) * NW * C
E1_PAD = _round_chunks(E1) * NW * C


def _make_seg_sum(n_dst_pad, epw, n_chunks, stage_rows=None):
    assert n_chunks % 3 == 0
    rpt = n_dst_pad // NS
    if stage_rows is not None:
        rb = (stage_rows // NS) // 8 * 8
        rb_last = stage_rows - (NS - 1) * rb

    mesh = plsc.VectorSubcoreMesh(core_axis_name="c", subcore_axis_name="s")

    @functools.partial(
        pl.kernel,
        out_type=(
            jax.ShapeDtypeStruct((NC * n_dst_pad, D), jnp.float32),
            jax.ShapeDtypeStruct((NW, n_dst_pad), jnp.float32),
        ),
        mesh=mesh,
        compiler_params=pltpu.CompilerParams(needs_layout_passes=False),
        scratch_types=[
            pltpu.VMEM((n_chunks, C), jnp.int32),
            pltpu.VMEM((n_chunks, C), jnp.int32),
            pltpu.VMEM((C, D), jnp.float32),
            pltpu.VMEM((C, D), jnp.float32),
            pltpu.VMEM((C, D), jnp.float32),
            pltpu.VMEM((n_dst_pad,), jnp.float32),
            pltpu.VMEM_SHARED((n_dst_pad, D), jnp.float32),
            (pltpu.VMEM_SHARED((stage_rows, D), jnp.float32)
             if stage_rows is not None else
             pltpu.VMEM((8,), jnp.float32)),
            pltpu.SemaphoreType.DMA,
            pltpu.SemaphoreType.DMA,
            pltpu.SemaphoreType.DMA,
            pltpu.SemaphoreType.DMA,
            pltpu.SemaphoreType.DMA,
            pltpu.SemaphoreType.DMA,
        ],
    )
    def seg(table, src, dst, zrows, zflat, sums_out, cnts_out,
            sidx, didx, rows0, rows1, rows2, cnt, acc, table_sp,
            gs0, gs1, gs2, ss0, ss1, ss2):
        rows = (rows0, rows1, rows2)
        gs = (gs0, gs1, gs2)
        ss = (ss0, ss1, ss2)
        c = lax.axis_index("c")
        s = lax.axis_index("s")
        wid = s * NC + c
        r0 = s * rpt

        pltpu.sync_copy(src.at[wid], sidx)
        pltpu.sync_copy(dst.at[wid], didx)
        pltpu.sync_copy(zrows.at[pl.ds(r0, rpt)], acc.at[pl.ds(r0, rpt)])
        pltpu.sync_copy(zflat.at[pl.ds(0, n_dst_pad)], cnt)
        if stage_rows is not None:
            @pl.when(s < NS - 1)
            def _():
                pltpu.sync_copy(table.at[pl.ds(s * rb, rb)],
                                table_sp.at[pl.ds(s * rb, rb)])

            @pl.when(s == NS - 1)
            def _():
                pltpu.sync_copy(table.at[pl.ds((NS - 1) * rb, rb_last)],
                                table_sp.at[pl.ds((NS - 1) * rb, rb_last)])
        plsc.subcore_barrier()

        gather_src = table_sp if stage_rows is not None else table
        ones16 = jnp.ones((16,), jnp.float32)

        def wait_gather(b):
            pltpu.make_async_copy(table.at[pl.ds(0, C)], rows[b], gs[b]).wait()

        def wait_scatter(b):
            pltpu.make_async_copy(table.at[pl.ds(0, C)], rows[b], ss[b]).wait()

        pltpu.async_copy(gather_src.at[sidx.at[0]], rows[0], gs[0])
        pltpu.async_copy(gather_src.at[sidx.at[1]], rows[1], gs[1])

        def body(t, carry):
            for b in (0, 1, 2):
                k = t * 3 + b
                wait_gather(b)
                for i in range(C // 16):
                    d = didx[k, pl.ds(i * 16, 16)]
                    plsc.addupdate_scatter(cnt, [d], ones16)
                pltpu.async_copy(rows[b], acc.at[didx.at[k]], ss[b], add=True)
                bb = (b + 2) % 3
                if b == 0:
                    @pl.when(t > 0)
                    def _():
                        wait_scatter(bb)
                else:
                    wait_scatter(bb)
                kn = lax.rem(k + 2, n_chunks)
                pltpu.async_copy(gather_src.at[sidx.at[kn]], rows[bb], gs[bb])
            return carry

        lax.fori_loop(0, n_chunks // 3, body, 0)
        wait_scatter(2)
        wait_gather(0)
        wait_gather(1)
        plsc.subcore_barrier()

        pltpu.sync_copy(acc.at[pl.ds(r0, rpt)],
                        sums_out.at[pl.ds(c * n_dst_pad + r0, rpt)])
        pltpu.sync_copy(cnt, cnts_out.at[wid])

    return seg


_seg0 = _make_seg_sum(N1_PAD, E0_PAD // NW, E0_PAD // (NW * C))
_seg1 = _make_seg_sum(N2_PAD, E1_PAD // NW, E1_PAD // (NW * C), stage_rows=N1)


def _make_tc_layer(n_dst, n_dst_pad, final):
    def body(sums_ref, cnts_ref, xt_ref, wl_ref, bl_ref, wr_ref, g_ref, b_ref,
             out_ref):
        ssum = (sums_ref[pl.ds(0, n_dst), :]
                + sums_ref[pl.ds(n_dst_pad, n_dst), :])
        cnt = jnp.sum(cnts_ref[:, :n_dst], axis=0)
        mean = ssum / jnp.maximum(cnt, 1.0)[:, None]
        xt = xt_ref[pl.ds(0, n_dst), :]
        z = (jnp.dot(mean, wl_ref[...], preferred_element_type=jnp.float32)
             + bl_ref[...][None, :]
             + jnp.dot(xt, wr_ref[...], preferred_element_type=jnp.float32))
        h = jnp.maximum(z, 0.0)
        m = jnp.mean(h, axis=0)
        v = jnp.mean((h - m[None, :]) ** 2, axis=0)
        hn = (h - m[None, :]) / jnp.sqrt(v + 1e-5)[None, :] * g_ref[...][None, :] \
            + b_ref[...][None, :]
        if final:
            mx = jnp.max(hn, axis=1, keepdims=True)
            e = hn - mx
            lse = jnp.log(jnp.sum(jnp.exp(e), axis=1, keepdims=True))
            out_ref[...] = e - lse
        else:
            out_ref[...] = hn

    return pl.pallas_call(
        body,
        out_shape=jax.ShapeDtypeStruct((n_dst, D), jnp.float32),
    )


_tc0 = _make_tc_layer(N1, N1_PAD, final=False)
_tc1 = _make_tc_layer(N2, N2_PAD, final=True)


def _pad_edges(ei, e_pad, n_src, n_dst, n_dst_pad):
    npad = e_pad - ei.shape[1]
    n_dummy = n_dst_pad - n_dst
    pad_src = np.arange(npad, dtype=np.int32) % n_src
    pad_dst = n_dst + np.arange(npad, dtype=np.int32) % n_dummy
    n_chunks = e_pad // (NW * C)
    src = jnp.concatenate([ei[0], jnp.asarray(pad_src)]).reshape(NW, n_chunks, C)
    dst = jnp.concatenate([ei[1], jnp.asarray(pad_dst)]).reshape(NW, n_chunks, C)
    return src, dst


def kernel(x, edge_index0, edge_index1,
           Wl0p, bl0p, Wr0p, g0p, b0p, Wl1p, bl1p, Wr1p, g1p, b1p,
           Wl0e, bl0e, Wr0e, g0e, b0e, Wl1e, bl1e, Wr1e, g1e, b1e):
    src0, dst0 = _pad_edges(edge_index0, E0_PAD, N0, N1, N1_PAD)
    src1, dst1 = _pad_edges(edge_index1, E1_PAD, N1, N2, N2_PAD)
    z0r = jnp.zeros((N1_PAD, D), jnp.float32)
    z0f = jnp.zeros((N1_PAD,), jnp.float32)
    z1r = jnp.zeros((N2_PAD, D), jnp.float32)
    z1f = jnp.zeros((N2_PAD,), jnp.float32)

    sums0, cnts0 = _seg0(x, src0, dst0, z0r, z0f)
    h1 = _tc0(sums0, cnts0, x, Wl0e, bl0e, Wr0e, g0e, b0e)
    sums1, cnts1 = _seg1(h1, src1, dst1, z1r, z1f)
    return _tc1(sums1, cnts1, h1, Wl1e, bl1e, Wr1e, g1e, b1e)


# speedup vs baseline: 1.0240x; 1.0240x over previous
import functools

import jax
import jax.numpy as jnp
import numpy as np
from jax import lax
from jax.experimental import pallas as pl
from jax.experimental.pallas import tpu as pltpu
from jax.experimental.pallas import tpu_sc as plsc

N0, N1, N2 = 10000, 5000, 2000
E0, E1 = 320000, 160000
D = 128

NC, NS = 2, 16
NW = NC * NS
C = 128

N1_PAD = 5120
N2_PAD = 2048
def _round_chunks(e):
    n = (e + NW * C - 1) // (NW * C)
    return n + (-n) % 3

E0_PAD = _round_chunks(E0) * NW * C
E1_PAD = _round_chunks(E1) * NW * C


def _make_seg_sum(n_dst_pad, epw, n_chunks, stage_rows=None):
    assert n_chunks % 3 == 0
    rpt = n_dst_pad // NS
    if stage_rows is not None:
        rb = (stage_rows // NS) // 8 * 8
        rb_last = stage_rows - (NS - 1) * rb

    mesh = plsc.VectorSubcoreMesh(core_axis_name="c", subcore_axis_name="s")

    @functools.partial(
        pl.kernel,
        out_type=(
            jax.ShapeDtypeStruct((NC * n_dst_pad, D), jnp.float32),
            jax.ShapeDtypeStruct((NW, n_dst_pad), jnp.float32),
        ),
        mesh=mesh,
        compiler_params=pltpu.CompilerParams(needs_layout_passes=False),
        scratch_types=[
            pltpu.VMEM((2 * n_chunks, C // 2), jnp.int32),
            pltpu.VMEM((n_chunks, C), jnp.int32),
            pltpu.VMEM((C, D), jnp.float32),
            pltpu.VMEM((C, D), jnp.float32),
            pltpu.VMEM((C, D), jnp.float32),
            pltpu.VMEM((n_dst_pad,), jnp.float32),
            pltpu.VMEM_SHARED((n_dst_pad, D), jnp.float32),
            (pltpu.VMEM_SHARED((stage_rows, D), jnp.float32)
             if stage_rows is not None else
             pltpu.VMEM((8,), jnp.float32)),
            pltpu.SemaphoreType.DMA,
            pltpu.SemaphoreType.DMA,
            pltpu.SemaphoreType.DMA,
            pltpu.SemaphoreType.DMA,
            pltpu.SemaphoreType.DMA,
            pltpu.SemaphoreType.DMA,
        ],
    )
    def seg(table, src, dst, zrows, zflat, sums_out, cnts_out,
            sidx, didx, rows0, rows1, rows2, cnt, acc, table_sp,
            gs0, gs1, gs2, ss0, ss1, ss2):
        rows = (rows0, rows1, rows2)
        gs = (gs0, gs1, gs2)
        ss = (ss0, ss1, ss2)
        c = lax.axis_index("c")
        s = lax.axis_index("s")
        wid = s * NC + c
        r0 = s * rpt

        pltpu.sync_copy(src.at[wid], sidx)
        pltpu.sync_copy(dst.at[wid], didx)
        pltpu.sync_copy(zrows.at[pl.ds(r0, rpt)], acc.at[pl.ds(r0, rpt)])
        pltpu.sync_copy(zflat.at[pl.ds(0, n_dst_pad)], cnt)
        if stage_rows is not None:
            @pl.when(s < NS - 1)
            def _():
                pltpu.sync_copy(table.at[pl.ds(s * rb, rb)],
                                table_sp.at[pl.ds(s * rb, rb)])

            @pl.when(s == NS - 1)
            def _():
                pltpu.sync_copy(table.at[pl.ds((NS - 1) * rb, rb_last)],
                                table_sp.at[pl.ds((NS - 1) * rb, rb_last)])
        plsc.subcore_barrier()

        gather_src = table_sp if stage_rows is not None else table
        ones16 = jnp.ones((16,), jnp.float32)

        def issue_gather(k, b):
            h = C // 2
            pltpu.async_copy(gather_src.at[sidx.at[2 * k]],
                             rows[b].at[pl.ds(0, h)], gs[b])
            pltpu.async_copy(gather_src.at[sidx.at[2 * k + 1]],
                             rows[b].at[pl.ds(h, h)], gs[b])

        def wait_gather(b):
            pltpu.make_async_copy(table.at[pl.ds(0, C)], rows[b], gs[b]).wait()

        def wait_scatter(b):
            pltpu.make_async_copy(table.at[pl.ds(0, C)], rows[b], ss[b]).wait()

        issue_gather(0, 0)
        issue_gather(1, 1)

        def body(t, carry):
            for b in (0, 1, 2):
                k = t * 3 + b
                wait_gather(b)
                for i in range(C // 16):
                    d = didx[k, pl.ds(i * 16, 16)]
                    plsc.addupdate_scatter(cnt, [d], ones16)
                pltpu.async_copy(rows[b], acc.at[didx.at[k]], ss[b], add=True)
                bb = (b + 2) % 3
                if b == 0:
                    @pl.when(t > 0)
                    def _():
                        wait_scatter(bb)
                else:
                    wait_scatter(bb)
                kn = lax.rem(k + 2, n_chunks)
                issue_gather(kn, bb)
            return carry

        lax.fori_loop(0, n_chunks // 3, body, 0)
        wait_scatter(2)
        wait_gather(0)
        wait_gather(1)
        plsc.subcore_barrier()

        pltpu.sync_copy(acc.at[pl.ds(r0, rpt)],
                        sums_out.at[pl.ds(c * n_dst_pad + r0, rpt)])
        pltpu.sync_copy(cnt, cnts_out.at[wid])

    return seg


_seg0 = _make_seg_sum(N1_PAD, E0_PAD // NW, E0_PAD // (NW * C))
_seg1 = _make_seg_sum(N2_PAD, E1_PAD // NW, E1_PAD // (NW * C))


def _make_tc_layer(n_dst, n_dst_pad, final):
    def body(sums_ref, cnts_ref, xt_ref, wl_ref, bl_ref, wr_ref, g_ref, b_ref,
             out_ref):
        ssum = (sums_ref[pl.ds(0, n_dst), :]
                + sums_ref[pl.ds(n_dst_pad, n_dst), :])
        cnt = jnp.sum(cnts_ref[:, :n_dst], axis=0)
        mean = ssum / jnp.maximum(cnt, 1.0)[:, None]
        xt = xt_ref[pl.ds(0, n_dst), :]
        z = (jnp.dot(mean, wl_ref[...], preferred_element_type=jnp.float32)
             + bl_ref[...][None, :]
             + jnp.dot(xt, wr_ref[...], preferred_element_type=jnp.float32))
        h = jnp.maximum(z, 0.0)
        m = jnp.mean(h, axis=0)
        v = jnp.mean((h - m[None, :]) ** 2, axis=0)
        hn = (h - m[None, :]) / jnp.sqrt(v + 1e-5)[None, :] * g_ref[...][None, :] \
            + b_ref[...][None, :]
        if final:
            mx = jnp.max(hn, axis=1, keepdims=True)
            e = hn - mx
            lse = jnp.log(jnp.sum(jnp.exp(e), axis=1, keepdims=True))
            out_ref[...] = e - lse
        else:
            out_ref[...] = hn

    return pl.pallas_call(
        body,
        out_shape=jax.ShapeDtypeStruct((n_dst, D), jnp.float32),
    )


_tc0 = _make_tc_layer(N1, N1_PAD, final=False)
_tc1 = _make_tc_layer(N2, N2_PAD, final=True)


def _pad_edges(ei, e_pad, n_src, n_dst, n_dst_pad):
    npad = e_pad - ei.shape[1]
    n_dummy = n_dst_pad - n_dst
    pad_src = np.arange(npad, dtype=np.int32) % n_src
    pad_dst = n_dst + np.arange(npad, dtype=np.int32) % n_dummy
    n_chunks = e_pad // (NW * C)
    src = jnp.concatenate([ei[0], jnp.asarray(pad_src)]) \
        .reshape(NW, 2 * n_chunks, C // 2)
    dst = jnp.concatenate([ei[1], jnp.asarray(pad_dst)]).reshape(NW, n_chunks, C)
    return src, dst


def kernel(x, edge_index0, edge_index1,
           Wl0p, bl0p, Wr0p, g0p, b0p, Wl1p, bl1p, Wr1p, g1p, b1p,
           Wl0e, bl0e, Wr0e, g0e, b0e, Wl1e, bl1e, Wr1e, g1e, b1e):
    src0, dst0 = _pad_edges(edge_index0, E0_PAD, N0, N1, N1_PAD)
    src1, dst1 = _pad_edges(edge_index1, E1_PAD, N1, N2, N2_PAD)
    z0r = jnp.zeros((N1_PAD, D), jnp.float32)
    z0f = jnp.zeros((N1_PAD,), jnp.float32)
    z1r = jnp.zeros((N2_PAD, D), jnp.float32)
    z1f = jnp.zeros((N2_PAD,), jnp.float32)

    sums0, cnts0 = _seg0(x, src0, dst0, z0r, z0f)
    h1 = _tc0(sums0, cnts0, x, Wl0e, bl0e, Wr0e, g0e, b0e)
    sums1, cnts1 = _seg1(h1, src1, dst1, z1r, z1f)
    return _tc1(sums1, cnts1, h1, Wl1e, bl1e, Wr1e, g1e, b1e)


# speedup vs baseline: 1.0363x; 1.0120x over previous
import functools

import jax
import jax.numpy as jnp
import numpy as np
from jax import lax
from jax.experimental import pallas as pl
from jax.experimental.pallas import tpu as pltpu
from jax.experimental.pallas import tpu_sc as plsc

N0, N1, N2 = 10000, 5000, 2000
E0, E1 = 320000, 160000
D = 128

NC, NS = 2, 16
NW = NC * NS
C = 128

N1_PAD = 5120
N2_PAD = 2048
def _round_chunks(e):
    n = (e + NW * C - 1) // (NW * C)
    return n + (-n) % 3

E0_PAD = _round_chunks(E0) * NW * C
E1_PAD = _round_chunks(E1) * NW * C


def _make_seg_sum(n_dst_pad, epw, n_chunks, stage_rows=None):
    assert n_chunks % 3 == 0
    rpt = n_dst_pad // NS
    if stage_rows is not None:
        rb = (stage_rows // NS) // 8 * 8
        rb_last = stage_rows - (NS - 1) * rb

    mesh = plsc.VectorSubcoreMesh(core_axis_name="c", subcore_axis_name="s")

    @functools.partial(
        pl.kernel,
        out_type=(
            jax.ShapeDtypeStruct((NC * n_dst_pad, D), jnp.float32),
            jax.ShapeDtypeStruct((NW, n_dst_pad), jnp.float32),
        ),
        mesh=mesh,
        compiler_params=pltpu.CompilerParams(needs_layout_passes=False),
        scratch_types=[
            pltpu.VMEM((n_chunks, C), jnp.int32),
            pltpu.VMEM((n_chunks, C), jnp.int32),
            pltpu.VMEM((C, D), jnp.float32),
            pltpu.VMEM((C, D), jnp.float32),
            pltpu.VMEM((C, D), jnp.float32),
            pltpu.VMEM((n_dst_pad,), jnp.float32),
            pltpu.VMEM_SHARED((n_dst_pad, D), jnp.float32),
            (pltpu.VMEM_SHARED((stage_rows, D), jnp.float32)
             if stage_rows is not None else
             pltpu.VMEM((8,), jnp.float32)),
            pltpu.SemaphoreType.DMA,
            pltpu.SemaphoreType.DMA,
            pltpu.SemaphoreType.DMA,
            pltpu.SemaphoreType.DMA,
            pltpu.SemaphoreType.DMA,
            pltpu.SemaphoreType.DMA,
        ],
    )
    def seg(table, src, dst, zrows, zflat, sums_out, cnts_out,
            sidx, didx, rows0, rows1, rows2, cnt, acc, table_sp,
            gs0, gs1, gs2, ss0, ss1, ss2):
        rows = (rows0, rows1, rows2)
        gs = (gs0, gs1, gs2)
        ss = (ss0, ss1, ss2)
        c = lax.axis_index("c")
        s = lax.axis_index("s")
        wid = s * NC + c
        r0 = s * rpt

        pltpu.sync_copy(src.at[wid], sidx)
        pltpu.sync_copy(dst.at[wid], didx)
        pltpu.sync_copy(zrows.at[pl.ds(r0, rpt)], acc.at[pl.ds(r0, rpt)])
        pltpu.sync_copy(zflat.at[pl.ds(0, n_dst_pad)], cnt)
        if stage_rows is not None:
            @pl.when(s < NS - 1)
            def _():
                pltpu.sync_copy(table.at[pl.ds(s * rb, rb)],
                                table_sp.at[pl.ds(s * rb, rb)])

            @pl.when(s == NS - 1)
            def _():
                pltpu.sync_copy(table.at[pl.ds((NS - 1) * rb, rb_last)],
                                table_sp.at[pl.ds((NS - 1) * rb, rb_last)])
        plsc.subcore_barrier()

        gather_src = table_sp if stage_rows is not None else table
        ones16 = jnp.ones((16,), jnp.float32)

        def issue_gather(k, b):
            pltpu.async_copy(gather_src.at[sidx.at[k]], rows[b], gs[b])

        def wait_gather(b):
            pltpu.make_async_copy(table.at[pl.ds(0, C)], rows[b], gs[b]).wait()

        def wait_scatter(b):
            pltpu.make_async_copy(table.at[pl.ds(0, C)], rows[b], ss[b]).wait()

        issue_gather(0, 0)
        issue_gather(1, 1)

        def body(t, carry):
            for b in (0, 1, 2):
                k = t * 3 + b
                wait_gather(b)
                for i in range(C // 16):
                    d = didx[k, pl.ds(i * 16, 16)]
                    plsc.addupdate_scatter(cnt, [d], ones16)
                pltpu.async_copy(rows[b], acc.at[didx.at[k]], ss[b], add=True)
                bb = (b + 2) % 3
                if b == 0:
                    @pl.when(t > 0)
                    def _():
                        wait_scatter(bb)
                else:
                    wait_scatter(bb)
                kn = lax.rem(k + 2, n_chunks)
                issue_gather(kn, bb)
            return carry

        lax.fori_loop(0, n_chunks // 3, body, 0)
        wait_scatter(2)
        wait_gather(0)
        wait_gather(1)
        plsc.subcore_barrier()

        pltpu.sync_copy(acc.at[pl.ds(r0, rpt)],
                        sums_out.at[pl.ds(c * n_dst_pad + r0, rpt)])
        pltpu.sync_copy(cnt, cnts_out.at[wid])

    return seg


_seg0 = _make_seg_sum(N1_PAD, E0_PAD // NW, E0_PAD // (NW * C))
_seg1 = _make_seg_sum(N2_PAD, E1_PAD // NW, E1_PAD // (NW * C))


def _make_tc_layer(n_dst, n_dst_pad, final):
    def body(sums_ref, cnts_ref, xt_ref, wl_ref, bl_ref, wr_ref, g_ref, b_ref,
             out_ref):
        ssum = (sums_ref[pl.ds(0, n_dst), :]
                + sums_ref[pl.ds(n_dst_pad, n_dst), :])
        cnt = jnp.sum(cnts_ref[:, :n_dst], axis=0)
        mean = ssum / jnp.maximum(cnt, 1.0)[:, None]
        xt = xt_ref[pl.ds(0, n_dst), :]
        z = (jnp.dot(mean, wl_ref[...], preferred_element_type=jnp.float32)
             + bl_ref[...][None, :]
             + jnp.dot(xt, wr_ref[...], preferred_element_type=jnp.float32))
        h = jnp.maximum(z, 0.0)
        m = jnp.mean(h, axis=0)
        v = jnp.mean((h - m[None, :]) ** 2, axis=0)
        hn = (h - m[None, :]) / jnp.sqrt(v + 1e-5)[None, :] * g_ref[...][None, :] \
            + b_ref[...][None, :]
        if final:
            mx = jnp.max(hn, axis=1, keepdims=True)
            e = hn - mx
            lse = jnp.log(jnp.sum(jnp.exp(e), axis=1, keepdims=True))
            out_ref[...] = e - lse
        else:
            out_ref[...] = hn

    return pl.pallas_call(
        body,
        out_shape=jax.ShapeDtypeStruct((n_dst, D), jnp.float32),
    )


_tc0 = _make_tc_layer(N1, N1_PAD, final=False)
_tc1 = _make_tc_layer(N2, N2_PAD, final=True)


def _pad_edges(ei, e_pad, n_src, n_dst, n_dst_pad):
    npad = e_pad - ei.shape[1]
    n_dummy = n_dst_pad - n_dst
    pad_src = np.arange(npad, dtype=np.int32) % n_src
    pad_dst = n_dst + np.arange(npad, dtype=np.int32) % n_dummy
    n_chunks = e_pad // (NW * C)
    src = jnp.concatenate([ei[0], jnp.asarray(pad_src)]).reshape(NW, n_chunks, C)
    dst = jnp.concatenate([ei[1], jnp.asarray(pad_dst)]).reshape(NW, n_chunks, C)
    return src, dst


def kernel(x, edge_index0, edge_index1,
           Wl0p, bl0p, Wr0p, g0p, b0p, Wl1p, bl1p, Wr1p, g1p, b1p,
           Wl0e, bl0e, Wr0e, g0e, b0e, Wl1e, bl1e, Wr1e, g1e, b1e):
    src0, dst0 = _pad_edges(edge_index0, E0_PAD, N0, N1, N1_PAD)
    src1, dst1 = _pad_edges(edge_index1, E1_PAD, N1, N2, N2_PAD)
    z0r = jnp.zeros((N1_PAD, D), jnp.float32)
    z0f = jnp.zeros((N1_PAD,), jnp.float32)
    z1r = jnp.zeros((N2_PAD, D), jnp.float32)
    z1f = jnp.zeros((N2_PAD,), jnp.float32)

    sums0, cnts0 = _seg0(x, src0, dst0, z0r, z0f)
    h1 = _tc0(sums0, cnts0, x, Wl0e, bl0e, Wr0e, g0e, b0e)
    sums1, cnts1 = _seg1(h1, src1, dst1, z1r, z1f)
    return _tc1(sums1, cnts1, h1, Wl1e, bl1e, Wr1e, g1e, b1e)


# speedup vs baseline: 1.0371x; 1.0007x over previous
import functools

import jax
import jax.numpy as jnp
import numpy as np
from jax import lax
from jax.experimental import pallas as pl
from jax.experimental.pallas import tpu as pltpu
from jax.experimental.pallas import tpu_sc as plsc

N0, N1, N2 = 10000, 5000, 2000
E0, E1 = 320000, 160000
D = 128

NC, NS = 2, 16
NW = NC * NS
C = 128

N1_PAD = 5120
N2_PAD = 2048
def _round_chunks(e):
    n = (e + NW * C - 1) // (NW * C)
    return n + (-n) % 3

E0_PAD = _round_chunks(E0) * NW * C
E1_PAD = _round_chunks(E1) * NW * C


def _make_seg_sum(n_dst_pad, n_chunks):
    assert n_chunks % 3 == 0
    rpt = n_dst_pad // NS

    mesh = plsc.VectorSubcoreMesh(core_axis_name="c", subcore_axis_name="s")

    @functools.partial(
        pl.kernel,
        out_type=(
            jax.ShapeDtypeStruct((NC * n_dst_pad, D), jnp.float32),
            jax.ShapeDtypeStruct((NW, n_dst_pad), jnp.float32),
        ),
        mesh=mesh,
        compiler_params=pltpu.CompilerParams(needs_layout_passes=False),
        scratch_types=[
            pltpu.VMEM((n_chunks, C), jnp.int32),
            pltpu.VMEM((n_chunks, C), jnp.int32),
            pltpu.VMEM((C, D), jnp.float32),
            pltpu.VMEM((C, D), jnp.float32),
            pltpu.VMEM((C, D), jnp.float32),
            pltpu.VMEM((n_dst_pad,), jnp.float32),
            pltpu.VMEM_SHARED((n_dst_pad, D), jnp.float32),
            pltpu.SemaphoreType.DMA,
            pltpu.SemaphoreType.DMA,
            pltpu.SemaphoreType.DMA,
            pltpu.SemaphoreType.DMA,
            pltpu.SemaphoreType.DMA,
            pltpu.SemaphoreType.DMA,
        ],
    )
    def seg(table, src, dst, zrows, zflat, sums_out, cnts_out,
            sidx, didx, rows0, rows1, rows2, cnt, acc,
            gs0, gs1, gs2, ss0, ss1, ss2):
        rows = (rows0, rows1, rows2)
        gs = (gs0, gs1, gs2)
        ss = (ss0, ss1, ss2)
        c = lax.axis_index("c")
        s = lax.axis_index("s")
        wid = s * NC + c
        r0 = s * rpt

        pltpu.sync_copy(src.at[wid], sidx)
        pltpu.sync_copy(dst.at[wid], didx)
        pltpu.sync_copy(zrows.at[pl.ds(r0, rpt)], acc.at[pl.ds(r0, rpt)])
        pltpu.sync_copy(zflat.at[pl.ds(0, n_dst_pad)], cnt)
        plsc.subcore_barrier()

        ones16 = jnp.ones((16,), jnp.float32)

        def issue_gather(k, b):
            pltpu.async_copy(table.at[sidx.at[k]], rows[b], gs[b])

        def wait_gather(b):
            pltpu.make_async_copy(table.at[pl.ds(0, C)], rows[b], gs[b]).wait()

        def wait_scatter(b):
            pltpu.make_async_copy(table.at[pl.ds(0, C)], rows[b], ss[b]).wait()

        issue_gather(0, 0)
        issue_gather(1, 1)

        def body(t, carry):
            for b in (0, 1, 2):
                k = t * 3 + b
                wait_gather(b)
                for i in range(C // 16):
                    d = didx[k, pl.ds(i * 16, 16)]
                    plsc.addupdate_scatter(cnt, [d], ones16)
                pltpu.async_copy(rows[b], acc.at[didx.at[k]], ss[b], add=True)
                bb = (b + 2) % 3
                if b == 0:
                    @pl.when(t > 0)
                    def _():
                        wait_scatter(bb)
                else:
                    wait_scatter(bb)
                kn = lax.rem(k + 2, n_chunks)
                issue_gather(kn, bb)
            return carry

        lax.fori_loop(0, n_chunks // 3, body, 0)
        wait_scatter(2)
        wait_gather(0)
        wait_gather(1)
        plsc.subcore_barrier()

        pltpu.sync_copy(acc.at[pl.ds(r0, rpt)],
                        sums_out.at[pl.ds(c * n_dst_pad + r0, rpt)])
        pltpu.sync_copy(cnt, cnts_out.at[wid])

    return seg


_seg0 = _make_seg_sum(N1_PAD, E0_PAD // (NW * C))
_seg1 = _make_seg_sum(N2_PAD, E1_PAD // (NW * C))


def _make_tc_layer(n_dst, n_dst_pad, final):
    def body(sums_ref, cnts_ref, xt_ref, wl_ref, bl_ref, wr_ref, g_ref, b_ref,
             out_ref):
        ssum = (sums_ref[pl.ds(0, n_dst), :]
                + sums_ref[pl.ds(n_dst_pad, n_dst), :])
        cnt = jnp.sum(cnts_ref[:, :n_dst], axis=0)
        mean = ssum / jnp.maximum(cnt, 1.0)[:, None]
        xt = xt_ref[pl.ds(0, n_dst), :]
        z = (jnp.dot(mean, wl_ref[...], preferred_element_type=jnp.float32)
             + bl_ref[...][None, :]
             + jnp.dot(xt, wr_ref[...], preferred_element_type=jnp.float32))
        h = jnp.maximum(z, 0.0)
        m = jnp.mean(h, axis=0)
        v = jnp.mean((h - m[None, :]) ** 2, axis=0)
        hn = (h - m[None, :]) / jnp.sqrt(v + 1e-5)[None, :] * g_ref[...][None, :] \
            + b_ref[...][None, :]
        if final:
            mx = jnp.max(hn, axis=1, keepdims=True)
            e = hn - mx
            lse = jnp.log(jnp.sum(jnp.exp(e), axis=1, keepdims=True))
            out_ref[...] = e - lse
        else:
            out_ref[...] = hn

    return pl.pallas_call(
        body,
        out_shape=jax.ShapeDtypeStruct((n_dst, D), jnp.float32),
    )


_tc0 = _make_tc_layer(N1, N1_PAD, final=False)
_tc1 = _make_tc_layer(N2, N2_PAD, final=True)


def _pad_edges(ei, e_pad, n_src, n_dst, n_dst_pad):
    npad = e_pad - ei.shape[1]
    n_dummy = n_dst_pad - n_dst
    pad_src = np.arange(npad, dtype=np.int32) % n_src
    pad_dst = n_dst + np.arange(npad, dtype=np.int32) % n_dummy
    n_chunks = e_pad // (NW * C)
    src = jnp.concatenate([ei[0], jnp.asarray(pad_src)]).reshape(NW, n_chunks, C)
    dst = jnp.concatenate([ei[1], jnp.asarray(pad_dst)]).reshape(NW, n_chunks, C)
    return src, dst


def kernel(x, edge_index0, edge_index1,
           Wl0p, bl0p, Wr0p, g0p, b0p, Wl1p, bl1p, Wr1p, g1p, b1p,
           Wl0e, bl0e, Wr0e, g0e, b0e, Wl1e, bl1e, Wr1e, g1e, b1e):
    src0, dst0 = _pad_edges(edge_index0, E0_PAD, N0, N1, N1_PAD)
    src1, dst1 = _pad_edges(edge_index1, E1_PAD, N1, N2, N2_PAD)
    z0r = jnp.zeros((N1_PAD, D), jnp.float32)
    z0f = jnp.zeros((N1_PAD,), jnp.float32)
    z1r = jnp.zeros((N2_PAD, D), jnp.float32)
    z1f = jnp.zeros((N2_PAD,), jnp.float32)

    sums0, cnts0 = _seg0(x, src0, dst0, z0r, z0f)
    h1 = _tc0(sums0, cnts0, x, Wl0e, bl0e, Wr0e, g0e, b0e)
    sums1, cnts1 = _seg1(h1, src1, dst1, z1r, z1f)
    return _tc1(sums1, cnts1, h1, Wl1e, bl1e, Wr1e, g1e, b1e)
